# baseline (speedup 1.0000x reference)
import functools

import jax
import jax.numpy as jnp
from jax import lax
from jax.experimental import pallas as pl
from jax.experimental.pallas import tpu as pltpu
from jax.experimental.pallas import tpu_sc as plsc

NCLS = 3
NBINS = 8192
H = 512
W = 512
NBATCH = 16
NWORK = 32
ROWS_W = NBATCH * H // NWORK
CHR = 16
NCH = ROWS_W // CHR
LANES = 16
VECS = CHR * W // LANES
VPR = W // LANES


def _sc_hist_kernel(x_hbm, t_hbm, out_hbm, xb0, xb1, xb2, lbb, h0, h1, h2):
    wid = lax.axis_index("s") * 2 + lax.axis_index("c")
    b = wid // 2
    half = wid % 2
    hists = (h0, h1, h2)

    def zero_body(i, carry):
        z = jnp.zeros((LANES,), jnp.float32)
        for hr in hists:
            hr[pl.ds(i * LANES, LANES)] = z
        return carry

    lax.fori_loop(0, 2 * NBINS // LANES, zero_body, 0)

    row0 = half * ROWS_W
    ones = jnp.ones((LANES,), jnp.float32)
    scale = jnp.full((LANES,), float(NBINS), jnp.float32)
    top = jnp.full((LANES,), NBINS - 1, jnp.int32)
    bot = jnp.zeros((LANES,), jnp.int32)
    fgoff = jnp.full((LANES,), NBINS, jnp.int32)
    izero = jnp.zeros((LANES,), jnp.int32)

    def chunk_body(k, carry):
        r0 = row0 + k * CHR
        pltpu.sync_copy(t_hbm.at[pl.ds(b, 1), pl.ds(r0, CHR), :], lbb)
        pltpu.sync_copy(x_hbm.at[pl.ds(b, 1), pl.ds(0, 1), pl.ds(r0, CHR), :],
                        xb0)
        pltpu.sync_copy(x_hbm.at[pl.ds(b, 1), pl.ds(1, 1), pl.ds(r0, CHR), :],
                        xb1)
        pltpu.sync_copy(x_hbm.at[pl.ds(b, 1), pl.ds(2, 1), pl.ds(r0, CHR), :],
                        xb2)

        def row_body(r, c2):
            for j in range(VPR):
                col = j * LANES
                v0 = xb0[0, 0, r, pl.ds(col, LANES)]
                v1 = xb1[0, 0, r, pl.ds(col, LANES)]
                v2 = xb2[0, 0, r, pl.ds(col, LANES)]
                t = lbb[0, r, pl.ds(col, LANES)]
                e0 = jnp.exp(v0)
                e1 = jnp.exp(v1)
                e2 = jnp.exp(v2)
                zsum = e0 + e1 + e2
                rs = scale / zsum
                for c, (ec, hr) in enumerate(((e0, h0), (e1, h1), (e2, h2))):
                    fgm = t == c
                    num = jnp.where(fgm, zsum - ec, ec)
                    bidx = (num * rs).astype(jnp.int32)
                    bidx = jnp.minimum(bidx, top)
                    bidx = bidx + jnp.where(fgm, fgoff, izero)
                    plsc.addupdate_scatter(hr, [bidx], ones)
            return c2

        lax.fori_loop(0, CHR, row_body, 0)
        return carry

    lax.fori_loop(0, NCH, chunk_body, 0)

    for c, hr in enumerate(hists):
        pltpu.sync_copy(hr.at[pl.ds(0, NBINS)], out_hbm.at[wid, 2 * c])
        pltpu.sync_copy(hr.at[pl.ds(NBINS, NBINS)],
                        out_hbm.at[wid, 2 * c + 1])


def _tc_scan_kernel(hist_ref, out_ref):
    hs = jnp.sum(hist_ref[...], axis=0)
    rows = NBINS // 128
    ri = lax.broadcasted_iota(jnp.int32, (128, 128), 0)
    ci = lax.broadcasted_iota(jnp.int32, (128, 128), 1)
    tri = (ri <= ci).astype(jnp.float32)
    rl = lax.broadcasted_iota(jnp.int32, (rows, rows), 0)
    cl = lax.broadcasted_iota(jnp.int32, (rows, rows), 1)
    low = (cl < rl).astype(jnp.float32)

    total = jnp.zeros((1, 1), jnp.float32)
    count = jnp.zeros((1, 1), jnp.float32)
    for c in range(NCLS):
        m = hs[2 * c + 1].reshape(rows, 128)
        n = hs[2 * c].reshape(rows, 128) + m
        csn = jnp.dot(n, tri, precision=lax.Precision.HIGHEST)
        csm = jnp.dot(m, tri, precision=lax.Precision.HIGHEST)
        offn = jnp.dot(low, csn[:, 127:128], precision=lax.Precision.HIGHEST)
        offm = jnp.dot(low, csm[:, 127:128], precision=lax.Precision.HIGHEST)
        csn = csn + offn
        csm = csm + offm
        ntot = jnp.sum(n)
        stot = jnp.sum(m)
        rn = ntot - csn + n
        rm = stot - csm + m
        denom = stot + rn - rm
        j_b = jnp.where(rn > 0.0, rn / jnp.where(denom > 0.0, denom, 1.0),
                        0.0)
        loss_c = (jnp.sum(j_b) - 0.5) / float(NBINS)
        inc = jnp.where(jnp.logical_or(stot > 0.0, c == 0), 1.0, 0.0)
        total = total + loss_c * inc
        count = count + inc
    out_ref[...] = total / jnp.maximum(count, 1.0)


_sc_hist = functools.partial(
    pl.kernel,
    mesh=plsc.VectorSubcoreMesh(core_axis_name="c", subcore_axis_name="s"),
    out_type=jax.ShapeDtypeStruct((NWORK, 2 * NCLS, NBINS), jnp.float32),
    compiler_params=pltpu.CompilerParams(needs_layout_passes=False),
    scratch_types=[
        pltpu.VMEM((1, 1, CHR, W), jnp.float32),
        pltpu.VMEM((1, 1, CHR, W), jnp.float32),
        pltpu.VMEM((1, 1, CHR, W), jnp.float32),
        pltpu.VMEM((1, CHR, W), jnp.int32),
        pltpu.VMEM((2 * NBINS,), jnp.float32),
        pltpu.VMEM((2 * NBINS,), jnp.float32),
        pltpu.VMEM((2 * NBINS,), jnp.float32),
    ],
)(_sc_hist_kernel)


_tc_scan = pl.pallas_call(
    _tc_scan_kernel,
    out_shape=jax.ShapeDtypeStruct((1, 1), jnp.float32),
)


def kernel(inputs, targets):
    hists = _sc_hist(inputs, targets)
    out = _tc_scan(hists)
    return out[0, 0]


# speedup vs baseline: 1.1994x; 1.1994x over previous
import functools

import jax
import jax.numpy as jnp
from jax import lax
from jax.experimental import pallas as pl
from jax.experimental.pallas import tpu as pltpu
from jax.experimental.pallas import tpu_sc as plsc

NCLS = 3
NBINS = 8192
H = 512
W = 512
NBATCH = 16
NWORK = 32
ROWS_W = NBATCH * H // NWORK
CHR = 16
NCH = ROWS_W // CHR
LANES = 16
VECS = CHR * W // LANES
VPR = W // LANES


def _sc_hist_kernel(x_hbm, t_hbm, out_hbm,
                    xa0, xa1, xa2, la, xb0, xb1, xb2, lb,
                    h0, h1, h2,
                    sa0, sa1, sa2, sal, sb0, sb1, sb2, sbl):
    wid = lax.axis_index("s") * 2 + lax.axis_index("c")
    b = wid // 2
    half = wid % 2
    hists = (h0, h1, h2)
    bufsets = ((xa0, xa1, xa2, la), (xb0, xb1, xb2, lb))
    semsets = ((sa0, sa1, sa2, sal), (sb0, sb1, sb2, sbl))

    def zero_body(i, carry):
        z = jnp.zeros((LANES,), jnp.float32)
        for hr in hists:
            hr[pl.ds(i * LANES, LANES)] = z
        return carry

    lax.fori_loop(0, 2 * NBINS // LANES, zero_body, 0)

    row0 = half * ROWS_W
    ones = jnp.ones((LANES,), jnp.float32)
    scale = jnp.full((LANES,), float(NBINS), jnp.float32)
    top = jnp.full((LANES,), NBINS - 1, jnp.int32)
    fgoff = jnp.full((LANES,), NBINS, jnp.int32)
    izero = jnp.zeros((LANES,), jnp.int32)

    def start(k, s):
        r0 = row0 + k * CHR
        bufs, sems = bufsets[s], semsets[s]
        hnds = [
            pltpu.async_copy(
                x_hbm.at[pl.ds(b, 1), pl.ds(c, 1), pl.ds(r0, CHR), :],
                bufs[c], sems[c])
            for c in range(NCLS)
        ]
        hnds.append(pltpu.async_copy(
            t_hbm.at[pl.ds(b, 1), pl.ds(r0, CHR), :], bufs[3], sems[3]))
        return hnds

    pending = start(0, 0)
    for k in range(NCH):
        s = k % 2
        nxt = start(k + 1, 1 - s) if k + 1 < NCH else None
        for hnd in pending:
            hnd.wait()
        c0, c1, c2, lbl = bufsets[s]

        def body(i, carry):
            r = i // VPR
            col = (i % VPR) * LANES
            v0 = c0[0, 0, r, pl.ds(col, LANES)]
            v1 = c1[0, 0, r, pl.ds(col, LANES)]
            v2 = c2[0, 0, r, pl.ds(col, LANES)]
            t = lbl[0, r, pl.ds(col, LANES)]
            e0 = jnp.exp(v0)
            e1 = jnp.exp(v1)
            e2 = jnp.exp(v2)
            zsum = e0 + e1 + e2
            rs = scale / zsum
            for c, (ec, hr) in enumerate(((e0, h0), (e1, h1), (e2, h2))):
                fgm = t == c
                num = jnp.where(fgm, zsum - ec, ec)
                bidx = (num * rs).astype(jnp.int32)
                bidx = jnp.minimum(bidx, top)
                bidx = bidx + jnp.where(fgm, fgoff, izero)
                plsc.addupdate_scatter(hr, [bidx], ones)
            return carry

        lax.fori_loop(0, VECS, body, 0)
        pending = nxt

    for c, hr in enumerate(hists):
        pltpu.sync_copy(hr.at[pl.ds(0, NBINS)], out_hbm.at[wid, 2 * c])
        pltpu.sync_copy(hr.at[pl.ds(NBINS, NBINS)],
                        out_hbm.at[wid, 2 * c + 1])


def _tc_scan_kernel(hist_ref, out_ref):
    hs = jnp.sum(hist_ref[...], axis=0)
    rows = NBINS // 128
    ri = lax.broadcasted_iota(jnp.int32, (128, 128), 0)
    ci = lax.broadcasted_iota(jnp.int32, (128, 128), 1)
    tri = (ri <= ci).astype(jnp.float32)
    rl = lax.broadcasted_iota(jnp.int32, (rows, rows), 0)
    cl = lax.broadcasted_iota(jnp.int32, (rows, rows), 1)
    low = (cl < rl).astype(jnp.float32)

    total = jnp.zeros((1, 1), jnp.float32)
    count = jnp.zeros((1, 1), jnp.float32)
    for c in range(NCLS):
        m = hs[2 * c + 1].reshape(rows, 128)
        n = hs[2 * c].reshape(rows, 128) + m
        csn = jnp.dot(n, tri, precision=lax.Precision.HIGHEST)
        csm = jnp.dot(m, tri, precision=lax.Precision.HIGHEST)
        offn = jnp.dot(low, csn[:, 127:128], precision=lax.Precision.HIGHEST)
        offm = jnp.dot(low, csm[:, 127:128], precision=lax.Precision.HIGHEST)
        csn = csn + offn
        csm = csm + offm
        ntot = jnp.sum(n)
        stot = jnp.sum(m)
        rn = ntot - csn + n
        rm = stot - csm + m
        denom = stot + rn - rm
        j_b = jnp.where(rn > 0.0, rn / jnp.where(denom > 0.0, denom, 1.0),
                        0.0)
        loss_c = (jnp.sum(j_b) - 0.5) / float(NBINS)
        inc = jnp.where(jnp.logical_or(stot > 0.0, c == 0), 1.0, 0.0)
        total = total + loss_c * inc
        count = count + inc
    out_ref[...] = total / jnp.maximum(count, 1.0)


_sc_hist = functools.partial(
    pl.kernel,
    mesh=plsc.VectorSubcoreMesh(core_axis_name="c", subcore_axis_name="s"),
    out_type=jax.ShapeDtypeStruct((NWORK, 2 * NCLS, NBINS), jnp.float32),
    compiler_params=pltpu.CompilerParams(needs_layout_passes=False),
    scratch_types=[
        pltpu.VMEM((1, 1, CHR, W), jnp.float32),
        pltpu.VMEM((1, 1, CHR, W), jnp.float32),
        pltpu.VMEM((1, 1, CHR, W), jnp.float32),
        pltpu.VMEM((1, CHR, W), jnp.int32),
        pltpu.VMEM((1, 1, CHR, W), jnp.float32),
        pltpu.VMEM((1, 1, CHR, W), jnp.float32),
        pltpu.VMEM((1, 1, CHR, W), jnp.float32),
        pltpu.VMEM((1, CHR, W), jnp.int32),
        pltpu.VMEM((2 * NBINS,), jnp.float32),
        pltpu.VMEM((2 * NBINS,), jnp.float32),
        pltpu.VMEM((2 * NBINS,), jnp.float32),
        pltpu.SemaphoreType.DMA,
        pltpu.SemaphoreType.DMA,
        pltpu.SemaphoreType.DMA,
        pltpu.SemaphoreType.DMA,
        pltpu.SemaphoreType.DMA,
        pltpu.SemaphoreType.DMA,
        pltpu.SemaphoreType.DMA,
        pltpu.SemaphoreType.DMA,
    ],
)(_sc_hist_kernel)


_tc_scan = pl.pallas_call(
    _tc_scan_kernel,
    out_shape=jax.ShapeDtypeStruct((1, 1), jnp.float32),
)


def kernel(inputs, targets):
    hists = _sc_hist(inputs, targets)
    out = _tc_scan(hists)
    return out[0, 0]


# speedup vs baseline: 2.7143x; 2.2631x over previous
import functools

import jax
import jax.numpy as jnp
from jax import lax
from jax.experimental import pallas as pl
from jax.experimental.pallas import tpu as pltpu
from jax.experimental.pallas import tpu_sc as plsc

NCLS = 3
NBINS = 8192
H = 512
W = 512
NBATCH = 16
NWORK = 32
ROWS_W = NBATCH * H // NWORK
CHR = 16
NCH = ROWS_W // CHR
LANES = 16
VECS = CHR * W // LANES
VPR = W // LANES
UN = 8
GPR = VPR // UN


def _sc_hist_kernel(x_hbm, t_hbm, out_hbm,
                    xa0, xa1, xa2, la, xb0, xb1, xb2, lb,
                    h0, h1, h2,
                    sa0, sa1, sa2, sal, sb0, sb1, sb2, sbl):
    wid = lax.axis_index("s") * 2 + lax.axis_index("c")
    b = wid // 2
    half = wid % 2
    hists = (h0, h1, h2)
    bufsets = ((xa0, xa1, xa2, la), (xb0, xb1, xb2, lb))
    semsets = ((sa0, sa1, sa2, sal), (sb0, sb1, sb2, sbl))

    def zero_body(i, carry):
        z = jnp.zeros((LANES,), jnp.float32)
        for hr in hists:
            hr[pl.ds(i * LANES, LANES)] = z
        return carry

    lax.fori_loop(0, 2 * NBINS // LANES, zero_body, 0)

    row0 = half * ROWS_W
    ones = jnp.ones((LANES,), jnp.float32)
    scale = jnp.full((LANES,), float(NBINS), jnp.float32)
    top = jnp.full((LANES,), NBINS - 1, jnp.int32)
    fgoff = jnp.full((LANES,), NBINS, jnp.int32)
    izero = jnp.zeros((LANES,), jnp.int32)

    def start(k, s):
        r0 = row0 + k * CHR
        bufs, sems = bufsets[s], semsets[s]
        hnds = [
            pltpu.async_copy(
                x_hbm.at[pl.ds(b, 1), pl.ds(c, 1), pl.ds(r0, CHR), :],
                bufs[c], sems[c])
            for c in range(NCLS)
        ]
        hnds.append(pltpu.async_copy(
            t_hbm.at[pl.ds(b, 1), pl.ds(r0, CHR), :], bufs[3], sems[3]))
        return hnds

    pending = start(0, 0)
    for k in range(NCH):
        s = k % 2
        nxt = start(k + 1, 1 - s) if k + 1 < NCH else None
        for hnd in pending:
            hnd.wait()
        c0, c1, c2, lbl = bufsets[s]

        def body(g, carry):
            r = g // GPR
            colb = (g % GPR) * (UN * LANES)
            cols = [colb + u * LANES for u in range(UN)]
            vs0 = [c0[0, 0, r, pl.ds(cc, LANES)] for cc in cols]
            vs1 = [c1[0, 0, r, pl.ds(cc, LANES)] for cc in cols]
            vs2 = [c2[0, 0, r, pl.ds(cc, LANES)] for cc in cols]
            ts = [lbl[0, r, pl.ds(cc, LANES)] for cc in cols]
            ex0 = [jnp.exp(v) for v in vs0]
            ex1 = [jnp.exp(v) for v in vs1]
            ex2 = [jnp.exp(v) for v in vs2]
            zs = [a + b + c for a, b, c in zip(ex0, ex1, ex2)]
            rss = [scale / z for z in zs]
            exs = (ex0, ex1, ex2)
            for c, hr in ((0, h0), (1, h1), (2, h2)):
                fgms = [t == c for t in ts]
                nums = [jnp.where(fgms[u], zs[u] - exs[c][u], exs[c][u])
                        for u in range(UN)]
                bidxs = [(nums[u] * rss[u]).astype(jnp.int32)
                         for u in range(UN)]
                bidxs = [jnp.minimum(bi, top) for bi in bidxs]
                bidxs = [bidxs[u] + jnp.where(fgms[u], fgoff, izero)
                         for u in range(UN)]
                for u in range(UN):
                    plsc.addupdate_scatter(hr, [bidxs[u]], ones)
            return carry

        lax.fori_loop(0, VECS // UN, body, 0)
        pending = nxt

    for c, hr in enumerate(hists):
        pltpu.sync_copy(hr.at[pl.ds(0, NBINS)], out_hbm.at[wid, 2 * c])
        pltpu.sync_copy(hr.at[pl.ds(NBINS, NBINS)],
                        out_hbm.at[wid, 2 * c + 1])


def _tc_scan_kernel(hist_ref, out_ref):
    hs = jnp.sum(hist_ref[...], axis=0)
    rows = NBINS // 128
    ri = lax.broadcasted_iota(jnp.int32, (128, 128), 0)
    ci = lax.broadcasted_iota(jnp.int32, (128, 128), 1)
    tri = (ri <= ci).astype(jnp.float32)
    rl = lax.broadcasted_iota(jnp.int32, (rows, rows), 0)
    cl = lax.broadcasted_iota(jnp.int32, (rows, rows), 1)
    low = (cl < rl).astype(jnp.float32)

    total = jnp.zeros((1, 1), jnp.float32)
    count = jnp.zeros((1, 1), jnp.float32)
    for c in range(NCLS):
        m = hs[2 * c + 1].reshape(rows, 128)
        n = hs[2 * c].reshape(rows, 128) + m
        csn = jnp.dot(n, tri, precision=lax.Precision.HIGHEST)
        csm = jnp.dot(m, tri, precision=lax.Precision.HIGHEST)
        offn = jnp.dot(low, csn[:, 127:128], precision=lax.Precision.HIGHEST)
        offm = jnp.dot(low, csm[:, 127:128], precision=lax.Precision.HIGHEST)
        csn = csn + offn
        csm = csm + offm
        ntot = jnp.sum(n)
        stot = jnp.sum(m)
        rn = ntot - csn + n
        rm = stot - csm + m
        denom = stot + rn - rm
        j_b = jnp.where(rn > 0.0, rn / jnp.where(denom > 0.0, denom, 1.0),
                        0.0)
        loss_c = (jnp.sum(j_b) - 0.5) / float(NBINS)
        inc = jnp.where(jnp.logical_or(stot > 0.0, c == 0), 1.0, 0.0)
        total = total + loss_c * inc
        count = count + inc
    out_ref[...] = total / jnp.maximum(count, 1.0)


_sc_hist = functools.partial(
    pl.kernel,
    mesh=plsc.VectorSubcoreMesh(core_axis_name="c", subcore_axis_name="s"),
    out_type=jax.ShapeDtypeStruct((NWORK, 2 * NCLS, NBINS), jnp.float32),
    compiler_params=pltpu.CompilerParams(needs_layout_passes=False),
    scratch_types=[
        pltpu.VMEM((1, 1, CHR, W), jnp.float32),
        pltpu.VMEM((1, 1, CHR, W), jnp.float32),
        pltpu.VMEM((1, 1, CHR, W), jnp.float32),
        pltpu.VMEM((1, CHR, W), jnp.int32),
        pltpu.VMEM((1, 1, CHR, W), jnp.float32),
        pltpu.VMEM((1, 1, CHR, W), jnp.float32),
        pltpu.VMEM((1, 1, CHR, W), jnp.float32),
        pltpu.VMEM((1, CHR, W), jnp.int32),
        pltpu.VMEM((2 * NBINS,), jnp.float32),
        pltpu.VMEM((2 * NBINS,), jnp.float32),
        pltpu.VMEM((2 * NBINS,), jnp.float32),
        pltpu.SemaphoreType.DMA,
        pltpu.SemaphoreType.DMA,
        pltpu.SemaphoreType.DMA,
        pltpu.SemaphoreType.DMA,
        pltpu.SemaphoreType.DMA,
        pltpu.SemaphoreType.DMA,
        pltpu.SemaphoreType.DMA,
        pltpu.SemaphoreType.DMA,
    ],
)(_sc_hist_kernel)


_tc_scan = pl.pallas_call(
    _tc_scan_kernel,
    out_shape=jax.ShapeDtypeStruct((1, 1), jnp.float32),
)


def kernel(inputs, targets):
    hists = _sc_hist(inputs, targets)
    out = _tc_scan(hists)
    return out[0, 0]


# speedup vs baseline: 2.9390x; 1.0828x over previous
import functools

import jax
import jax.numpy as jnp
from jax import lax
from jax.experimental import pallas as pl
from jax.experimental.pallas import tpu as pltpu
from jax.experimental.pallas import tpu_sc as plsc

NCLS = 3
NBINS = 8192
H = 512
W = 512
NBATCH = 16
NWORK = 32
ROWS_W = NBATCH * H // NWORK
CHR = 16
NCH = ROWS_W // CHR
LANES = 16
VECS = CHR * W // LANES
VPR = W // LANES
UN = 8
GPR = VPR // UN


def _sc_hist_kernel(x_hbm, t_hbm, out_hbm,
                    xa0, xa1, xa2, la, xb0, xb1, xb2, lb,
                    h0, h1, h2,
                    sa0, sa1, sa2, sal, sb0, sb1, sb2, sbl):
    wid = lax.axis_index("s") * 2 + lax.axis_index("c")
    b = wid // 2
    half = wid % 2
    hists = (h0, h1, h2)
    bufsets = ((xa0, xa1, xa2, la), (xb0, xb1, xb2, lb))
    semsets = ((sa0, sa1, sa2, sal), (sb0, sb1, sb2, sbl))

    def zero_body(i, carry):
        z = jnp.zeros((LANES,), jnp.float32)
        for hr in hists:
            hr[pl.ds(i * LANES, LANES)] = z
        return carry

    lax.fori_loop(0, 2 * NBINS // LANES, zero_body, 0)

    row0 = half * ROWS_W
    ones = jnp.ones((LANES,), jnp.float32)
    scale = jnp.full((LANES,), float(NBINS), jnp.float32)
    topf = jnp.full((LANES,), NBINS - 0.5, jnp.float32)
    revf = jnp.full((LANES,), 2 * NBINS - 0.00390625, jnp.float32)

    def start(k, s):
        r0 = row0 + k * CHR
        bufs, sems = bufsets[s], semsets[s]
        hnds = [
            pltpu.async_copy(
                x_hbm.at[pl.ds(b, 1), pl.ds(c, 1), pl.ds(r0, CHR), :],
                bufs[c], sems[c])
            for c in range(NCLS)
        ]
        hnds.append(pltpu.async_copy(
            t_hbm.at[pl.ds(b, 1), pl.ds(r0, CHR), :], bufs[3], sems[3]))
        return hnds

    pending = start(0, 0)
    for k in range(NCH):
        s = k % 2
        nxt = start(k + 1, 1 - s) if k + 1 < NCH else None
        for hnd in pending:
            hnd.wait()
        c0, c1, c2, lbl = bufsets[s]

        def body(g, carry):
            r = g // GPR
            colb = (g % GPR) * (UN * LANES)
            cols = [colb + u * LANES for u in range(UN)]
            vs0 = [c0[0, 0, r, pl.ds(cc, LANES)] for cc in cols]
            vs1 = [c1[0, 0, r, pl.ds(cc, LANES)] for cc in cols]
            vs2 = [c2[0, 0, r, pl.ds(cc, LANES)] for cc in cols]
            ts = [lbl[0, r, pl.ds(cc, LANES)] for cc in cols]
            ex0 = [jnp.exp(v) for v in vs0]
            ex1 = [jnp.exp(v) for v in vs1]
            ex2 = [jnp.exp(v) for v in vs2]
            zs = [a + b + c for a, b, c in zip(ex0, ex1, ex2)]
            rss = [scale / z for z in zs]
            exs = (ex0, ex1, ex2)
            for c, hr in ((0, h0), (1, h1), (2, h2)):
                fgms = [t == c for t in ts]
                qs = [exs[c][u] * rss[u] for u in range(UN)]
                qs = [jnp.minimum(q, topf) for q in qs]
                revs = [revf - q for q in qs]
                bsel = [jnp.where(fgms[u], revs[u], qs[u])
                        for u in range(UN)]
                bidxs = [bv.astype(jnp.int32) for bv in bsel]
                for u in range(UN):
                    plsc.addupdate_scatter(hr, [bidxs[u]], ones)
            return carry

        lax.fori_loop(0, VECS // UN, body, 0)
        pending = nxt

    for c, hr in enumerate(hists):
        pltpu.sync_copy(hr.at[pl.ds(0, NBINS)], out_hbm.at[wid, 2 * c])
        pltpu.sync_copy(hr.at[pl.ds(NBINS, NBINS)],
                        out_hbm.at[wid, 2 * c + 1])


def _tc_scan_kernel(hist_ref, out_ref):
    hs = jnp.sum(hist_ref[...], axis=0)
    rows = NBINS // 128
    ri = lax.broadcasted_iota(jnp.int32, (128, 128), 0)
    ci = lax.broadcasted_iota(jnp.int32, (128, 128), 1)
    tri = (ri <= ci).astype(jnp.float32)
    rl = lax.broadcasted_iota(jnp.int32, (rows, rows), 0)
    cl = lax.broadcasted_iota(jnp.int32, (rows, rows), 1)
    low = (cl < rl).astype(jnp.float32)
    total = jnp.zeros((1, 1), jnp.float32)
    count = jnp.zeros((1, 1), jnp.float32)
    for c in range(NCLS):
        m = hs[2 * c + 1].reshape(rows, 128)
        n = hs[2 * c].reshape(rows, 128) + m
        csn = jnp.dot(n, tri, precision=lax.Precision.HIGHEST)
        csm = jnp.dot(m, tri, precision=lax.Precision.HIGHEST)
        offn = jnp.dot(low, csn[:, 127:128], precision=lax.Precision.HIGHEST)
        offm = jnp.dot(low, csm[:, 127:128], precision=lax.Precision.HIGHEST)
        csn = csn + offn
        csm = csm + offm
        ntot = jnp.sum(n)
        stot = jnp.sum(m)
        rn = ntot - csn + n
        rm = stot - csm + m
        denom = stot + rn - rm
        j_b = jnp.where(rn > 0.0, rn / jnp.where(denom > 0.0, denom, 1.0),
                        0.0)
        loss_c = (jnp.sum(j_b) - 0.5) / float(NBINS)
        inc = jnp.where(jnp.logical_or(stot > 0.0, c == 0), 1.0, 0.0)
        total = total + loss_c * inc
        count = count + inc
    out_ref[...] = total / jnp.maximum(count, 1.0)


_sc_hist = functools.partial(
    pl.kernel,
    mesh=plsc.VectorSubcoreMesh(core_axis_name="c", subcore_axis_name="s"),
    out_type=jax.ShapeDtypeStruct((NWORK, 2 * NCLS, NBINS), jnp.float32),
    compiler_params=pltpu.CompilerParams(needs_layout_passes=False),
    scratch_types=[
        pltpu.VMEM((1, 1, CHR, W), jnp.float32),
        pltpu.VMEM((1, 1, CHR, W), jnp.float32),
        pltpu.VMEM((1, 1, CHR, W), jnp.float32),
        pltpu.VMEM((1, CHR, W), jnp.int32),
        pltpu.VMEM((1, 1, CHR, W), jnp.float32),
        pltpu.VMEM((1, 1, CHR, W), jnp.float32),
        pltpu.VMEM((1, 1, CHR, W), jnp.float32),
        pltpu.VMEM((1, CHR, W), jnp.int32),
        pltpu.VMEM((2 * NBINS,), jnp.float32),
        pltpu.VMEM((2 * NBINS,), jnp.float32),
        pltpu.VMEM((2 * NBINS,), jnp.float32),
        pltpu.SemaphoreType.DMA,
        pltpu.SemaphoreType.DMA,
        pltpu.SemaphoreType.DMA,
        pltpu.SemaphoreType.DMA,
        pltpu.SemaphoreType.DMA,
        pltpu.SemaphoreType.DMA,
        pltpu.SemaphoreType.DMA,
        pltpu.SemaphoreType.DMA,
    ],
)(_sc_hist_kernel)


_tc_scan = pl.pallas_call(
    _tc_scan_kernel,
    out_shape=jax.ShapeDtypeStruct((1, 1), jnp.float32),
)


def kernel(inputs, targets):
    hists = _sc_hist(inputs, targets)
    out = _tc_scan(hists)
    return out[0, 0]


# speedup vs baseline: 2.9779x; 1.0132x over previous
import functools

import jax
import jax.numpy as jnp
from jax import lax
from jax.experimental import pallas as pl
from jax.experimental.pallas import tpu as pltpu
from jax.experimental.pallas import tpu_sc as plsc

NCLS = 3
NBINS = 8192
H = 512
W = 512
NBATCH = 16
NWORK = 32
ROWS_W = NBATCH * H // NWORK
CHR = 16
NCH = ROWS_W // CHR
LANES = 16
VECS = CHR * W // LANES
VPR = W // LANES
UN = 16
GPR = VPR // UN


def _sc_hist_kernel(x_hbm, t_hbm, out_hbm,
                    xa0, xa1, xa2, la, xb0, xb1, xb2, lb,
                    h0, h1, h2,
                    sa0, sa1, sa2, sal, sb0, sb1, sb2, sbl):
    wid = lax.axis_index("s") * 2 + lax.axis_index("c")
    b = wid // 2
    half = wid % 2
    hists = (h0, h1, h2)
    bufsets = ((xa0, xa1, xa2, la), (xb0, xb1, xb2, lb))
    semsets = ((sa0, sa1, sa2, sal), (sb0, sb1, sb2, sbl))

    def zero_body(i, carry):
        z = jnp.zeros((LANES,), jnp.float32)
        for hr in hists:
            hr[pl.ds(i * LANES, LANES)] = z
        return carry

    lax.fori_loop(0, 2 * NBINS // LANES, zero_body, 0)

    row0 = half * ROWS_W
    ones = jnp.ones((LANES,), jnp.float32)
    scale = jnp.full((LANES,), float(NBINS), jnp.float32)
    topf = jnp.full((LANES,), NBINS - 0.5, jnp.float32)
    revf = jnp.full((LANES,), 2 * NBINS - 0.00390625, jnp.float32)

    def start(k, s):
        r0 = row0 + k * CHR
        bufs, sems = bufsets[s], semsets[s]
        hnds = [
            pltpu.async_copy(
                x_hbm.at[pl.ds(b, 1), pl.ds(c, 1), pl.ds(r0, CHR), :],
                bufs[c], sems[c])
            for c in range(NCLS)
        ]
        hnds.append(pltpu.async_copy(
            t_hbm.at[pl.ds(b, 1), pl.ds(r0, CHR), :], bufs[3], sems[3]))
        return hnds

    pending = start(0, 0)
    for k in range(NCH):
        s = k % 2
        nxt = start(k + 1, 1 - s) if k + 1 < NCH else None
        for hnd in pending:
            hnd.wait()
        c0, c1, c2, lbl = bufsets[s]

        def body(g, carry):
            r = g // GPR
            colb = (g % GPR) * (UN * LANES)
            cols = [colb + u * LANES for u in range(UN)]
            vs0 = [c0[0, 0, r, pl.ds(cc, LANES)] for cc in cols]
            vs1 = [c1[0, 0, r, pl.ds(cc, LANES)] for cc in cols]
            vs2 = [c2[0, 0, r, pl.ds(cc, LANES)] for cc in cols]
            ts = [lbl[0, r, pl.ds(cc, LANES)] for cc in cols]
            ex0 = [jnp.exp(v) for v in vs0]
            ex1 = [jnp.exp(v) for v in vs1]
            ex2 = [jnp.exp(v) for v in vs2]
            zs = [a + b + c for a, b, c in zip(ex0, ex1, ex2)]
            rss = [scale / z for z in zs]
            exs = (ex0, ex1, ex2)
            for c, hr in ((0, h0), (1, h1), (2, h2)):
                fgms = [t == c for t in ts]
                qs = [exs[c][u] * rss[u] for u in range(UN)]
                qs = [jnp.minimum(q, topf) for q in qs]
                revs = [revf - q for q in qs]
                bsel = [jnp.where(fgms[u], revs[u], qs[u])
                        for u in range(UN)]
                bidxs = [bv.astype(jnp.int32) for bv in bsel]
                for u in range(UN):
                    plsc.addupdate_scatter(hr, [bidxs[u]], ones)
            return carry

        lax.fori_loop(0, VECS // UN, body, 0)
        pending = nxt

    for c, hr in enumerate(hists):
        pltpu.sync_copy(hr.at[pl.ds(0, NBINS)], out_hbm.at[wid, 2 * c])
        pltpu.sync_copy(hr.at[pl.ds(NBINS, NBINS)],
                        out_hbm.at[wid, 2 * c + 1])


def _tc_scan_kernel(hist_ref, out_ref):
    hs = jnp.sum(hist_ref[...], axis=0)
    rows = NBINS // 128
    ri = lax.broadcasted_iota(jnp.int32, (128, 128), 0)
    ci = lax.broadcasted_iota(jnp.int32, (128, 128), 1)
    tri = (ri <= ci).astype(jnp.float32)
    rl = lax.broadcasted_iota(jnp.int32, (rows, rows), 0)
    cl = lax.broadcasted_iota(jnp.int32, (rows, rows), 1)
    low = (cl < rl).astype(jnp.float32)
    total = jnp.zeros((1, 1), jnp.float32)
    count = jnp.zeros((1, 1), jnp.float32)
    for c in range(NCLS):
        m = hs[2 * c + 1].reshape(rows, 128)
        n = hs[2 * c].reshape(rows, 128) + m
        csn = jnp.dot(n, tri, precision=lax.Precision.HIGHEST)
        csm = jnp.dot(m, tri, precision=lax.Precision.HIGHEST)
        offn = jnp.dot(low, csn[:, 127:128], precision=lax.Precision.HIGHEST)
        offm = jnp.dot(low, csm[:, 127:128], precision=lax.Precision.HIGHEST)
        csn = csn + offn
        csm = csm + offm
        ntot = jnp.sum(n)
        stot = jnp.sum(m)
        rn = ntot - csn + n
        rm = stot - csm + m
        denom = stot + rn - rm
        j_b = jnp.where(rn > 0.0, rn / jnp.where(denom > 0.0, denom, 1.0),
                        0.0)
        loss_c = (jnp.sum(j_b) - 0.5) / float(NBINS)
        inc = jnp.where(jnp.logical_or(stot > 0.0, c == 0), 1.0, 0.0)
        total = total + loss_c * inc
        count = count + inc
    out_ref[...] = total / jnp.maximum(count, 1.0)


_sc_hist = functools.partial(
    pl.kernel,
    mesh=plsc.VectorSubcoreMesh(core_axis_name="c", subcore_axis_name="s"),
    out_type=jax.ShapeDtypeStruct((NWORK, 2 * NCLS, NBINS), jnp.float32),
    compiler_params=pltpu.CompilerParams(needs_layout_passes=False),
    scratch_types=[
        pltpu.VMEM((1, 1, CHR, W), jnp.float32),
        pltpu.VMEM((1, 1, CHR, W), jnp.float32),
        pltpu.VMEM((1, 1, CHR, W), jnp.float32),
        pltpu.VMEM((1, CHR, W), jnp.int32),
        pltpu.VMEM((1, 1, CHR, W), jnp.float32),
        pltpu.VMEM((1, 1, CHR, W), jnp.float32),
        pltpu.VMEM((1, 1, CHR, W), jnp.float32),
        pltpu.VMEM((1, CHR, W), jnp.int32),
        pltpu.VMEM((2 * NBINS,), jnp.float32),
        pltpu.VMEM((2 * NBINS,), jnp.float32),
        pltpu.VMEM((2 * NBINS,), jnp.float32),
        pltpu.SemaphoreType.DMA,
        pltpu.SemaphoreType.DMA,
        pltpu.SemaphoreType.DMA,
        pltpu.SemaphoreType.DMA,
        pltpu.SemaphoreType.DMA,
        pltpu.SemaphoreType.DMA,
        pltpu.SemaphoreType.DMA,
        pltpu.SemaphoreType.DMA,
    ],
)(_sc_hist_kernel)


_tc_scan = pl.pallas_call(
    _tc_scan_kernel,
    out_shape=jax.ShapeDtypeStruct((1, 1), jnp.float32),
)


def kernel(inputs, targets):
    hists = _sc_hist(inputs, targets)
    out = _tc_scan(hists)
    return out[0, 0]
